# Initial kernel scaffold; baseline (speedup 1.0000x reference)
import jax
import jax.numpy as jnp
from jax.experimental import pallas as pl


def kernel(x, Wqk, bqk, Wv, bv, Wout, bout):
    raise NotImplementedError("write your pallas kernel here")


# speedup vs baseline: 10614.5381x; 10614.5381x over previous
import functools

import jax
import jax.numpy as jnp
from jax import lax
from jax.experimental import pallas as pl
from jax.experimental.pallas import tpu as pltpu

N_HASHES = 8
BUCKET = 64
NB = 32
TBLK = 256

_HIGH = lax.Precision.HIGHEST


def _argmax_first(vals, n):
    m = jnp.max(vals, axis=-1, keepdims=True)
    ids = lax.broadcasted_iota(jnp.int32, vals.shape, vals.ndim - 1)
    cand = jnp.where(vals >= m, ids, n)
    return jnp.min(cand, axis=-1)


def _stage_a(x_ref, wqk_ref, bqk_ref, wv_ref, bv_ref, rot_ref,
             qk_ref, qkn_ref, v_ref, bkt_ref):
    xb = x_ref[...].astype(jnp.bfloat16)
    qk = lax.dot_general(xb, wqk_ref[...].astype(jnp.bfloat16),
                         (((1,), (1,)), ((), ())),
                         preferred_element_type=jnp.float32) + bqk_ref[...]
    vv = lax.dot_general(xb, wv_ref[...].astype(jnp.bfloat16),
                         (((1,), (1,)), ((), ())),
                         preferred_element_type=jnp.float32) + bv_ref[...]
    qk_ref[...] = qk
    v_ref[...] = vv
    nrm = jnp.sqrt(jnp.sum(qk * qk, axis=1, keepdims=True))
    nrm = jnp.maximum(nrm, 1e-12)
    qkn_ref[...] = qk / nrm * (1.0 / 32.0)
    rv = jnp.dot(qk.astype(jnp.bfloat16),
                 rot_ref[...].astype(jnp.bfloat16),
                 preferred_element_type=jnp.float32)
    cols = []
    for h in range(N_HASHES):
        sl = rv[:, h * (NB // 2):(h + 1) * (NB // 2)]
        vals = jnp.concatenate([sl, -sl], axis=1)
        cols.append(_argmax_first(vals, NB).reshape(-1, 1))
    bkt_ref[...] = jnp.concatenate(cols, axis=1)


def _stage_c(bkt_ref, gch_ref):
    s = bkt_ref.shape[0]
    r = lax.broadcasted_iota(jnp.int32, (s, s), 0)
    c = lax.broadcasted_iota(jnp.int32, (s, s), 1)
    tril = (c < r).astype(jnp.bfloat16)
    r2 = lax.broadcasted_iota(jnp.int32, (NB, NB), 0)
    c2 = lax.broadcasted_iota(jnp.int32, (NB, NB), 1)
    m32 = (r2 < c2).astype(jnp.float32)
    lane = lax.broadcasted_iota(jnp.int32, (1, NB), 1)
    cols = []
    for h in range(N_HASHES):
        b = bkt_ref[...][:, h:h + 1]
        e = (b == lane).astype(jnp.float32)
        hist = jnp.sum(e, axis=0, keepdims=True)
        offs = jnp.dot(hist, m32, preferred_element_type=jnp.float32,
                       precision=_HIGH)
        cume = jnp.dot(tril, e.astype(jnp.bfloat16),
                       preferred_element_type=jnp.float32)
        rank = jnp.sum(cume * e, axis=1, keepdims=True)
        base = jnp.sum(e * offs, axis=1, keepdims=True)
        pos = (rank + base).astype(jnp.int32)
        cols.append((pos >> 6) + h * NB)
    gch_ref[...] = jnp.concatenate(cols, axis=1)


def _onehot_chunks(g):
    lane = lax.broadcasted_iota(jnp.int32, (1, N_HASHES * NB), 1)
    u = jnp.zeros((g.shape[0], N_HASHES * NB), jnp.float32)
    for h in range(N_HASHES):
        u = u + (g[:, h:h + 1] == lane).astype(jnp.float32)
    return u.astype(jnp.bfloat16)


def _stage_e(qk_ref, qkn_ref, v_ref, gi_ref, gj_ref, wout_ref, bout_ref,
             out_ref, acc_ref):
    j = pl.program_id(2)
    nj = pl.num_programs(2)

    @pl.when(j == 0)
    def _():
        acc_ref[...] = jnp.zeros_like(acc_ref)

    u_i = _onehot_chunks(gi_ref[...])
    u_j = _onehot_chunks(gj_ref[...])
    cb = lax.dot_general(u_i, u_j, (((1,), (1,)), ((), ())),
                         preferred_element_type=jnp.float32)
    sb = lax.dot_general(qk_ref[...].astype(jnp.bfloat16),
                         qkn_ref[...].astype(jnp.bfloat16),
                         (((1,), (1,)), ((), ())),
                         preferred_element_type=jnp.float32)
    w = (cb * sb).astype(jnp.bfloat16)
    acc_ref[...] += jnp.dot(w, v_ref[...].astype(jnp.bfloat16),
                            preferred_element_type=jnp.float32)

    @pl.when(j == nj - 1)
    def _():
        out_ref[...] = lax.dot_general(
            acc_ref[...].astype(jnp.bfloat16),
            wout_ref[...].astype(jnp.bfloat16),
            (((1,), (1,)), ((), ())),
            preferred_element_type=jnp.float32) + bout_ref[...]


def kernel(x, Wqk, bqk, Wv, bv, Wout, bout):
    B, S, D = x.shape
    nblk = S // TBLK
    rot = jax.random.normal(jax.random.key(42), (1, D, N_HASHES, NB // 2),
                            dtype=x.dtype)
    rot2 = rot.reshape(D, N_HASHES * (NB // 2))

    qk, qkn, v, bkt = pl.pallas_call(
        _stage_a,
        grid=(B, nblk),
        in_specs=[
            pl.BlockSpec((None, TBLK, D), lambda b, i: (b, i, 0)),
            pl.BlockSpec((D, D), lambda b, i: (0, 0)),
            pl.BlockSpec((1, D), lambda b, i: (0, 0)),
            pl.BlockSpec((D, D), lambda b, i: (0, 0)),
            pl.BlockSpec((1, D), lambda b, i: (0, 0)),
            pl.BlockSpec((D, N_HASHES * (NB // 2)), lambda b, i: (0, 0)),
        ],
        out_specs=[
            pl.BlockSpec((None, TBLK, D), lambda b, i: (b, i, 0)),
            pl.BlockSpec((None, TBLK, D), lambda b, i: (b, i, 0)),
            pl.BlockSpec((None, TBLK, D), lambda b, i: (b, i, 0)),
            pl.BlockSpec((None, TBLK, N_HASHES), lambda b, i: (b, i, 0)),
        ],
        out_shape=[
            jax.ShapeDtypeStruct((B, S, D), jnp.float32),
            jax.ShapeDtypeStruct((B, S, D), jnp.float32),
            jax.ShapeDtypeStruct((B, S, D), jnp.float32),
            jax.ShapeDtypeStruct((B, S, N_HASHES), jnp.int32),
        ],
    )(x, Wqk, bqk.reshape(1, D), Wv, bv.reshape(1, D), rot2)

    gch = pl.pallas_call(
        _stage_c,
        grid=(B,),
        in_specs=[pl.BlockSpec((None, S, N_HASHES), lambda b: (b, 0, 0))],
        out_specs=pl.BlockSpec((None, S, N_HASHES), lambda b: (b, 0, 0)),
        out_shape=jax.ShapeDtypeStruct((B, S, N_HASHES), jnp.int32),
    )(bkt)

    out = pl.pallas_call(
        _stage_e,
        grid=(B, nblk, nblk),
        in_specs=[
            pl.BlockSpec((None, TBLK, D), lambda b, i, j: (b, i, 0)),
            pl.BlockSpec((None, TBLK, D), lambda b, i, j: (b, j, 0)),
            pl.BlockSpec((None, TBLK, D), lambda b, i, j: (b, j, 0)),
            pl.BlockSpec((None, TBLK, N_HASHES), lambda b, i, j: (b, i, 0)),
            pl.BlockSpec((None, TBLK, N_HASHES), lambda b, i, j: (b, j, 0)),
            pl.BlockSpec((D, D), lambda b, i, j: (0, 0)),
            pl.BlockSpec((1, D), lambda b, i, j: (0, 0)),
        ],
        out_specs=pl.BlockSpec((None, TBLK, D), lambda b, i, j: (b, i, 0)),
        out_shape=jax.ShapeDtypeStruct((B, S, D), jnp.float32),
        scratch_shapes=[pltpu.VMEM((TBLK, D), jnp.float32)],
        compiler_params=pltpu.CompilerParams(
            dimension_semantics=("parallel", "parallel", "arbitrary")),
    )(qk, qkn, v, gch, gch, Wout, bout.reshape(1, D))

    return out


# speedup vs baseline: 11199.7322x; 1.0551x over previous
import functools

import jax
import jax.numpy as jnp
from jax import lax
from jax.experimental import pallas as pl
from jax.experimental.pallas import tpu as pltpu

N_HASHES = 8
BUCKET = 64
NB = 32
TBLK = 256

_HIGH = lax.Precision.HIGHEST


def _argmax_first(vals, n):
    m = jnp.max(vals, axis=-1, keepdims=True)
    ids = lax.broadcasted_iota(jnp.int32, vals.shape, vals.ndim - 1)
    cand = jnp.where(vals >= m, ids, n)
    return jnp.min(cand, axis=-1)


def _stage_a(x_ref, wqk_ref, bqk_ref, wv_ref, bv_ref, rot_ref,
             qk_ref, qkn_ref, v_ref, bkt_ref):
    xb = x_ref[...].astype(jnp.bfloat16)
    qk = lax.dot_general(xb, wqk_ref[...].astype(jnp.bfloat16),
                         (((1,), (1,)), ((), ())),
                         preferred_element_type=jnp.float32) + bqk_ref[...]
    vv = lax.dot_general(xb, wv_ref[...].astype(jnp.bfloat16),
                         (((1,), (1,)), ((), ())),
                         preferred_element_type=jnp.float32) + bv_ref[...]
    qk_ref[...] = qk
    v_ref[...] = vv
    nrm = jnp.sqrt(jnp.sum(qk * qk, axis=1, keepdims=True))
    nrm = jnp.maximum(nrm, 1e-12)
    qkn_ref[...] = qk / nrm * (1.0 / 32.0)
    rv = jnp.dot(qk.astype(jnp.bfloat16),
                 rot_ref[...].astype(jnp.bfloat16),
                 preferred_element_type=jnp.float32)
    cols = []
    for h in range(N_HASHES):
        sl = rv[:, h * (NB // 2):(h + 1) * (NB // 2)]
        vals = jnp.concatenate([sl, -sl], axis=1)
        cols.append(_argmax_first(vals, NB).reshape(-1, 1))
    bkt_ref[...] = jnp.concatenate(cols, axis=1)


def _stage_c(bkt_ref, gch_ref):
    s = bkt_ref.shape[0]
    nh, nb = N_HASHES, NB
    r = lax.broadcasted_iota(jnp.int32, (s, s), 0)
    c = lax.broadcasted_iota(jnp.int32, (s, s), 1)
    tril = (c < r).astype(jnp.bfloat16)
    r2 = lax.broadcasted_iota(jnp.int32, (nh * nb, nh * nb), 0)
    c2 = lax.broadcasted_iota(jnp.int32, (nh * nb, nh * nb), 1)
    mblk = ((r2 < c2) & ((r2 >> 5) == (c2 >> 5))).astype(jnp.float32)
    lane = lax.broadcasted_iota(jnp.int32, (1, nh * nb), 1)
    ball = bkt_ref[...]
    e = jnp.zeros((s, nh * nb), jnp.float32)
    for h in range(nh):
        e = e + (ball[:, h:h + 1] + h * nb == lane).astype(jnp.float32)
    hist = jnp.sum(e, axis=0, keepdims=True)
    offs = jnp.dot(hist, mblk, preferred_element_type=jnp.float32,
                   precision=_HIGH)
    cume = jnp.dot(tril, e.astype(jnp.bfloat16),
                   preferred_element_type=jnp.float32)
    tot = cume * e + e * offs
    cols = []
    for h in range(nh):
        pos = jnp.sum(tot[:, h * nb:(h + 1) * nb], axis=1,
                      keepdims=True).astype(jnp.int32)
        cols.append((pos >> 6) + h * nb)
    gch_ref[...] = jnp.concatenate(cols, axis=1)


def _onehot_chunks(g):
    lane = lax.broadcasted_iota(jnp.int32, (1, N_HASHES * NB), 1)
    u = jnp.zeros((g.shape[0], N_HASHES * NB), jnp.float32)
    for h in range(N_HASHES):
        u = u + (g[:, h:h + 1] == lane).astype(jnp.float32)
    return u.astype(jnp.bfloat16)


def _stage_e(qk_ref, qkn_ref, v_ref, gi_ref, gj_ref, wout_ref, bout_ref,
             out_ref, acc_ref):
    j = pl.program_id(2)
    nj = pl.num_programs(2)

    @pl.when(j == 0)
    def _():
        acc_ref[...] = jnp.zeros_like(acc_ref)

    u_i = _onehot_chunks(gi_ref[...])
    u_j = _onehot_chunks(gj_ref[...])
    cb = lax.dot_general(u_i, u_j, (((1,), (1,)), ((), ())),
                         preferred_element_type=jnp.float32)
    sb = lax.dot_general(qk_ref[...].astype(jnp.bfloat16),
                         qkn_ref[...].astype(jnp.bfloat16),
                         (((1,), (1,)), ((), ())),
                         preferred_element_type=jnp.float32)
    w = (cb * sb).astype(jnp.bfloat16)
    acc_ref[...] += jnp.dot(w, v_ref[...].astype(jnp.bfloat16),
                            preferred_element_type=jnp.float32)

    @pl.when(j == nj - 1)
    def _():
        out_ref[...] = lax.dot_general(
            acc_ref[...].astype(jnp.bfloat16),
            wout_ref[...].astype(jnp.bfloat16),
            (((1,), (1,)), ((), ())),
            preferred_element_type=jnp.float32) + bout_ref[...]


def kernel(x, Wqk, bqk, Wv, bv, Wout, bout):
    B, S, D = x.shape
    nblk = S // TBLK
    rot = jax.random.normal(jax.random.key(42), (1, D, N_HASHES, NB // 2),
                            dtype=x.dtype)
    rot2 = rot.reshape(D, N_HASHES * (NB // 2))

    qk, qkn, v, bkt = pl.pallas_call(
        _stage_a,
        grid=(B, nblk),
        in_specs=[
            pl.BlockSpec((None, TBLK, D), lambda b, i: (b, i, 0)),
            pl.BlockSpec((D, D), lambda b, i: (0, 0)),
            pl.BlockSpec((1, D), lambda b, i: (0, 0)),
            pl.BlockSpec((D, D), lambda b, i: (0, 0)),
            pl.BlockSpec((1, D), lambda b, i: (0, 0)),
            pl.BlockSpec((D, N_HASHES * (NB // 2)), lambda b, i: (0, 0)),
        ],
        out_specs=[
            pl.BlockSpec((None, TBLK, D), lambda b, i: (b, i, 0)),
            pl.BlockSpec((None, TBLK, D), lambda b, i: (b, i, 0)),
            pl.BlockSpec((None, TBLK, D), lambda b, i: (b, i, 0)),
            pl.BlockSpec((None, TBLK, N_HASHES), lambda b, i: (b, i, 0)),
        ],
        out_shape=[
            jax.ShapeDtypeStruct((B, S, D), jnp.float32),
            jax.ShapeDtypeStruct((B, S, D), jnp.float32),
            jax.ShapeDtypeStruct((B, S, D), jnp.float32),
            jax.ShapeDtypeStruct((B, S, N_HASHES), jnp.int32),
        ],
    )(x, Wqk, bqk.reshape(1, D), Wv, bv.reshape(1, D), rot2)

    gch = pl.pallas_call(
        _stage_c,
        grid=(B,),
        in_specs=[pl.BlockSpec((None, S, N_HASHES), lambda b: (b, 0, 0))],
        out_specs=pl.BlockSpec((None, S, N_HASHES), lambda b: (b, 0, 0)),
        out_shape=jax.ShapeDtypeStruct((B, S, N_HASHES), jnp.int32),
    )(bkt)

    out = pl.pallas_call(
        _stage_e,
        grid=(B, nblk, nblk),
        in_specs=[
            pl.BlockSpec((None, TBLK, D), lambda b, i, j: (b, i, 0)),
            pl.BlockSpec((None, TBLK, D), lambda b, i, j: (b, j, 0)),
            pl.BlockSpec((None, TBLK, D), lambda b, i, j: (b, j, 0)),
            pl.BlockSpec((None, TBLK, N_HASHES), lambda b, i, j: (b, i, 0)),
            pl.BlockSpec((None, TBLK, N_HASHES), lambda b, i, j: (b, j, 0)),
            pl.BlockSpec((D, D), lambda b, i, j: (0, 0)),
            pl.BlockSpec((1, D), lambda b, i, j: (0, 0)),
        ],
        out_specs=pl.BlockSpec((None, TBLK, D), lambda b, i, j: (b, i, 0)),
        out_shape=jax.ShapeDtypeStruct((B, S, D), jnp.float32),
        scratch_shapes=[pltpu.VMEM((TBLK, D), jnp.float32)],
        compiler_params=pltpu.CompilerParams(
            dimension_semantics=("parallel", "parallel", "arbitrary")),
    )(qk, qkn, v, gch, gch, Wout, bout.reshape(1, D))

    return out


# speedup vs baseline: 18257.1791x; 1.6301x over previous
import functools

import jax
import jax.numpy as jnp
from jax import lax
from jax.experimental import pallas as pl
from jax.experimental.pallas import tpu as pltpu

N_HASHES = 8
BUCKET = 64
NB = 32
TBLK = 256

_HIGH = lax.Precision.HIGHEST


def _argmax_first(vals, n):
    m = jnp.max(vals, axis=-1, keepdims=True)
    ids = lax.broadcasted_iota(jnp.int32, vals.shape, vals.ndim - 1)
    cand = jnp.where(vals >= m, ids, n)
    return jnp.min(cand, axis=-1)


def _stage_a(x_ref, wqk_ref, bqk_ref, wv_ref, bv_ref, rot_ref,
             qk_ref, rn_ref, v_ref, bkt_ref):
    xb = x_ref[...].astype(jnp.bfloat16)
    qk = lax.dot_general(xb, wqk_ref[...].astype(jnp.bfloat16),
                         (((1,), (1,)), ((), ())),
                         preferred_element_type=jnp.float32) + bqk_ref[...]
    vv = lax.dot_general(xb, wv_ref[...].astype(jnp.bfloat16),
                         (((1,), (1,)), ((), ())),
                         preferred_element_type=jnp.float32) + bv_ref[...]
    qk_ref[...] = qk
    v_ref[...] = vv
    nrm = jnp.sqrt(jnp.sum(qk * qk, axis=1, keepdims=True))
    nrm = jnp.maximum(nrm, 1e-12)
    rn_ref[...] = (1.0 / 32.0) / nrm
    rv = jnp.dot(qk.astype(jnp.bfloat16),
                 rot_ref[...].astype(jnp.bfloat16),
                 preferred_element_type=jnp.float32)
    cols = []
    for h in range(N_HASHES):
        sl = rv[:, h * (NB // 2):(h + 1) * (NB // 2)]
        vals = jnp.concatenate([sl, -sl], axis=1)
        cols.append(_argmax_first(vals, NB).reshape(-1, 1))
    bkt_ref[...] = jnp.concatenate(cols, axis=1)


def _stage_c(bkt_ref, gch_ref):
    s = bkt_ref.shape[0]
    nh, nb = N_HASHES, NB
    r = lax.broadcasted_iota(jnp.int32, (s, s), 0)
    c = lax.broadcasted_iota(jnp.int32, (s, s), 1)
    tril = (c < r).astype(jnp.bfloat16)
    r2 = lax.broadcasted_iota(jnp.int32, (nh * nb, nh * nb), 0)
    c2 = lax.broadcasted_iota(jnp.int32, (nh * nb, nh * nb), 1)
    mblk = ((r2 < c2) & ((r2 >> 5) == (c2 >> 5))).astype(jnp.float32)
    lane = lax.broadcasted_iota(jnp.int32, (1, nh * nb), 1)
    ball = bkt_ref[...]
    e = jnp.zeros((s, nh * nb), jnp.float32)
    for h in range(nh):
        e = e + (ball[:, h:h + 1] + h * nb == lane).astype(jnp.float32)
    hist = jnp.sum(e, axis=0, keepdims=True)
    offs = jnp.dot(hist, mblk, preferred_element_type=jnp.float32,
                   precision=_HIGH)
    cume = jnp.dot(tril, e.astype(jnp.bfloat16),
                   preferred_element_type=jnp.float32)
    tot = cume * e + e * offs
    cols = []
    for h in range(nh):
        pos = jnp.sum(tot[:, h * nb:(h + 1) * nb], axis=1,
                      keepdims=True).astype(jnp.int32)
        cols.append((pos >> 6) + h * nb)
    gch_ref[...] = jnp.concatenate(cols, axis=1)


def _onehot_chunks(g):
    lane = lax.broadcasted_iota(jnp.int32, (1, N_HASHES * NB), 1)
    u = jnp.zeros((g.shape[0], N_HASHES * NB), jnp.float32)
    for h in range(N_HASHES):
        u = u + (g[:, h:h + 1] == lane).astype(jnp.float32)
    return u.astype(jnp.bfloat16)


def _stage_e(qki_ref, qkf_ref, rn_ref, v_ref, gi_ref, gf_ref,
             wout_ref, bout_ref, out_ref):
    nj = qkf_ref.shape[0] // TBLK
    u_i = _onehot_chunks(gi_ref[...])
    qki_bf = qki_ref[...].astype(jnp.bfloat16)
    acc = jnp.zeros((TBLK, qkf_ref.shape[1]), jnp.float32)
    for j in range(nj):
        sl = slice(j * TBLK, (j + 1) * TBLK)
        u_j = _onehot_chunks(gf_ref[sl, :])
        cb = lax.dot_general(u_i, u_j, (((1,), (1,)), ((), ())),
                             preferred_element_type=jnp.float32)
        qkn_j = (qkf_ref[sl, :] * rn_ref[sl, :]).astype(jnp.bfloat16)
        sb = lax.dot_general(qki_bf, qkn_j, (((1,), (1,)), ((), ())),
                             preferred_element_type=jnp.float32)
        w = (cb * sb).astype(jnp.bfloat16)
        acc = acc + jnp.dot(w, v_ref[sl, :].astype(jnp.bfloat16),
                            preferred_element_type=jnp.float32)
    out_ref[...] = lax.dot_general(
        acc.astype(jnp.bfloat16), wout_ref[...].astype(jnp.bfloat16),
        (((1,), (1,)), ((), ())),
        preferred_element_type=jnp.float32) + bout_ref[...]


def kernel(x, Wqk, bqk, Wv, bv, Wout, bout):
    B, S, D = x.shape
    nblk = S // TBLK
    rot = jax.random.normal(jax.random.key(42), (1, D, N_HASHES, NB // 2),
                            dtype=x.dtype)
    rot2 = rot.reshape(D, N_HASHES * (NB // 2))

    qk, rn, v, bkt = pl.pallas_call(
        _stage_a,
        grid=(B, nblk),
        in_specs=[
            pl.BlockSpec((None, TBLK, D), lambda b, i: (b, i, 0)),
            pl.BlockSpec((D, D), lambda b, i: (0, 0)),
            pl.BlockSpec((1, D), lambda b, i: (0, 0)),
            pl.BlockSpec((D, D), lambda b, i: (0, 0)),
            pl.BlockSpec((1, D), lambda b, i: (0, 0)),
            pl.BlockSpec((D, N_HASHES * (NB // 2)), lambda b, i: (0, 0)),
        ],
        out_specs=[
            pl.BlockSpec((None, TBLK, D), lambda b, i: (b, i, 0)),
            pl.BlockSpec((None, TBLK, 1), lambda b, i: (b, i, 0)),
            pl.BlockSpec((None, TBLK, D), lambda b, i: (b, i, 0)),
            pl.BlockSpec((None, TBLK, N_HASHES), lambda b, i: (b, i, 0)),
        ],
        out_shape=[
            jax.ShapeDtypeStruct((B, S, D), jnp.float32),
            jax.ShapeDtypeStruct((B, S, 1), jnp.float32),
            jax.ShapeDtypeStruct((B, S, D), jnp.float32),
            jax.ShapeDtypeStruct((B, S, N_HASHES), jnp.int32),
        ],
    )(x, Wqk, bqk.reshape(1, D), Wv, bv.reshape(1, D), rot2)

    gch = pl.pallas_call(
        _stage_c,
        grid=(B,),
        in_specs=[pl.BlockSpec((None, S, N_HASHES), lambda b: (b, 0, 0))],
        out_specs=pl.BlockSpec((None, S, N_HASHES), lambda b: (b, 0, 0)),
        out_shape=jax.ShapeDtypeStruct((B, S, N_HASHES), jnp.int32),
    )(bkt)

    out = pl.pallas_call(
        _stage_e,
        grid=(B, nblk),
        in_specs=[
            pl.BlockSpec((None, TBLK, D), lambda b, i: (b, i, 0)),
            pl.BlockSpec((None, S, D), lambda b, i: (b, 0, 0)),
            pl.BlockSpec((None, S, 1), lambda b, i: (b, 0, 0)),
            pl.BlockSpec((None, S, D), lambda b, i: (b, 0, 0)),
            pl.BlockSpec((None, TBLK, N_HASHES), lambda b, i: (b, i, 0)),
            pl.BlockSpec((None, S, N_HASHES), lambda b, i: (b, 0, 0)),
            pl.BlockSpec((D, D), lambda b, i: (0, 0)),
            pl.BlockSpec((1, D), lambda b, i: (0, 0)),
        ],
        out_specs=pl.BlockSpec((None, TBLK, D), lambda b, i: (b, i, 0)),
        out_shape=jax.ShapeDtypeStruct((B, S, D), jnp.float32),
        compiler_params=pltpu.CompilerParams(
            dimension_semantics=("parallel", "arbitrary")),
    )(qk, qk, rn, v, gch, gch, Wout, bout.reshape(1, D))

    return out
